# baseline (speedup 1.0000x reference)
import functools

import jax
import jax.numpy as jnp
from jax import lax
from jax.experimental import pallas as pl
from jax.experimental.pallas import tpu as pltpu
from jax.experimental.pallas import tpu_sc as plsc

_POOL_N = 500000
_MAX_K = 512
_NLOOP = 4
_HALT = 0.9
_D = 256
_VOCAB = 32000
_B = 4
_T = 512
_NTOK = _B * _T
_VB = 640
_WIN = 640


def _layer_norm(x, g, b):
    m = jnp.mean(x, axis=-1, keepdims=True)
    v = jnp.mean((x - m) ** 2, axis=-1, keepdims=True)
    return (x - m) / jnp.sqrt(v + 1e-6) * g + b


def _softplus(x):
    return jnp.maximum(x, 0.0) + jnp.log(1.0 + jnp.exp(-jnp.abs(x)))


def _sc_gather(table, idx):
    info = plsc.get_sparse_core_info()
    nw = info.num_cores * info.num_subcores
    n = idx.shape[0]
    bpw = n // nw
    mesh = plsc.VectorSubcoreMesh(core_axis_name="c", subcore_axis_name="s")

    @functools.partial(
        pl.kernel,
        mesh=mesh,
        out_type=jax.ShapeDtypeStruct((n, _D), jnp.float32),
        scratch_types=[
            pltpu.VMEM((bpw,), jnp.int32),
            pltpu.VMEM((bpw, _D), jnp.float32),
            pltpu.SemaphoreType.DMA,
        ],
    )
    def k(table_hbm, idx_hbm, out_hbm, idx_v, rows_v, sem):
        wid = lax.axis_index("s") * info.num_cores + lax.axis_index("c")
        base = wid * bpw
        pltpu.sync_copy(idx_hbm.at[pl.ds(base, bpw)], idx_v)
        pltpu.async_copy(table_hbm.at[idx_v], rows_v, sem).wait()
        pltpu.sync_copy(rows_v, out_hbm.at[pl.ds(base, bpw)])

    return k(table, idx)


def _bdot(a, b):
    return jnp.dot(a.astype(jnp.bfloat16), b.astype(jnp.bfloat16),
                   preferred_element_type=jnp.float32)


def _fused_body(h0_ref, we1, be1, we2, be2, lneg, lneb, widx, bidx,
                pool_ref, wi1, bi1, wi2, bi2, lnig, lnib, whalt, bh,
                wdec_ref, bdec_ref, logits_ref, idx_out, state_ref,
                win_ref, idxs_ref, sem):
  j = pl.program_id(0)

  @pl.when(j == 0)
  def _controller_step():
    h0 = h0_ref[...]
    pre = _bdot(h0, we1[...]) + be1[...]
    h = h0 + _bdot(jax.nn.gelu(pre), we2[...]) + be2[...]
    h = _layer_norm(h, lneg[...], lneb[...])

    state = h
    halt_prob = jnp.zeros((_NTOK, 1), jnp.float32)
    halted = jnp.zeros((_NTOK, 1), jnp.float32)
    jvec = lax.broadcasted_iota(jnp.int32, (1, _WIN), 1)
    starts_list = []

    for _ in range(_NLOOP):
        pooled = jnp.concatenate(
            [jnp.mean(state[b * _T:(b + 1) * _T, :], axis=0, keepdims=True)
             for b in range(_B)],
            axis=0)
        raw = _bdot(pooled, widx[...]) + bidx[...]
        mu = jax.nn.sigmoid(raw[:, 0:1])
        sigma = _softplus(raw[:, 1:2]) + 1e-3
        start_i = jnp.floor(mu * float(_POOL_N - _MAX_K)).astype(jnp.int32)
        starts_list.append(start_i)
        astart = jnp.minimum((start_i // 8) * 8, _POOL_N - _WIN)
        off = start_i - astart
        row = jnp.concatenate([astart, jnp.zeros((_B, 127), jnp.int32)],
                              axis=1)
        idxs_ref[...] = jnp.concatenate(
            [row, jnp.zeros((8 - _B, 128), jnp.int32)], axis=0)
        copies = []
        for b in range(_B):
            a_b = pl.multiple_of(idxs_ref[b, 0], 8)
            c = pltpu.make_async_copy(
                pool_ref.at[pl.ds(a_b, _WIN), :], win_ref.at[b], sem)
            c.start()
            copies.append(c)
        retrieved = []
        for b in range(_B):
            copies[b].wait()
            sig = sigma[b:b + 1, 0:1]
            k = jvec - off[b:b + 1, 0:1]
            valid = (k >= 0) & (k < _MAX_K)
            pos = k.astype(jnp.float32) / float(_MAX_K) - 0.5
            wlog = -(pos * pos) / (2.0 * sig * sig)
            wmax = jnp.max(jnp.where(valid, wlog, -jnp.inf), axis=-1,
                           keepdims=True)
            e = jnp.where(valid, jnp.exp(wlog - wmax), 0.0)
            w = e / jnp.sum(e, axis=-1, keepdims=True)
            retrieved.append(_bdot(w, win_ref[b]))
        r_full = jnp.concatenate(
            [jnp.broadcast_to(retrieved[b], (_T, _D)) for b in range(_B)],
            axis=0)
        comb = jnp.concatenate([state, r_full], axis=1)
        integ = _bdot(jax.nn.gelu(_bdot(comb, wi1[...]) + bi1[...]),
                      wi2[...]) + bi2[...]
        integ = _layer_norm(integ, lnig[...], lnib[...])
        cand = state + integ
        p = jax.nn.sigmoid(_bdot(cand, whalt[...]) + bh[...])
        hp_new = halt_prob + p * (1.0 - halted)
        new_halted = (hp_new >= _HALT).astype(jnp.float32)
        state = (1.0 - halted) * cand + halted * state
        halt_prob = hp_new
        halted = new_halted

    state_ref[...] = state.astype(jnp.bfloat16)
    idx_out[...] = jnp.concatenate(starts_list, axis=1)

  logits_ref[...] = jnp.dot(state_ref[...], wdec_ref[...].astype(jnp.bfloat16),
                            preferred_element_type=jnp.float32) + bdec_ref[...]


def _fused(h0, W_e1, b_e1, W_e2, b_e2, ln_e_g, ln_e_b, W_idx, b_idx2,
           pool, W_i1, b_i1, W_i2, b_i2, ln_i_g, ln_i_b, W_halt, b_halt2,
           W_dec, b_dec2):
    hbm = pl.BlockSpec(memory_space=pltpu.MemorySpace.HBM)
    in_specs = [pl.BlockSpec(x.shape, lambda j: (0,) * x.ndim)
                for x in (h0, W_e1, b_e1, W_e2, b_e2, ln_e_g, ln_e_b,
                          W_idx, b_idx2)]
    in_specs.append(hbm)
    in_specs += [pl.BlockSpec(x.shape, lambda j: (0,) * x.ndim)
                 for x in (W_i1, b_i1, W_i2, b_i2, ln_i_g, ln_i_b,
                           W_halt, b_halt2)]
    in_specs += [pl.BlockSpec((_D, _VB), lambda j: (0, j)),
                 pl.BlockSpec((1, _VB), lambda j: (0, j))]
    return pl.pallas_call(
        _fused_body,
        grid=(_VOCAB // _VB,),
        in_specs=in_specs,
        out_specs=[pl.BlockSpec((_NTOK, _VB), lambda j: (0, j)),
                   pl.BlockSpec((_B, _NLOOP), lambda j: (0, 0))],
        out_shape=[jax.ShapeDtypeStruct((_NTOK, _VOCAB), jnp.float32),
                   jax.ShapeDtypeStruct((_B, _NLOOP), jnp.int32)],
        scratch_shapes=[pltpu.VMEM((_NTOK, _D), jnp.bfloat16),
                        pltpu.VMEM((_B, _WIN, _D), jnp.float32),
                        pltpu.VMEM((8, 128), jnp.int32),
                        pltpu.SemaphoreType.DMA],
    )(h0, W_e1, b_e1, W_e2, b_e2, ln_e_g, ln_e_b, W_idx, b_idx2, pool,
      W_i1, b_i1, W_i2, b_i2, ln_i_g, ln_i_b, W_halt, b_halt2, W_dec,
      b_dec2)


def kernel(input_ids, embed, W_e1, b_e1, W_e2, b_e2, ln_e_g, ln_e_b, W_dec,
           b_dec, W_idx, b_idx, pool, W_i1, b_i1, W_i2, b_i2, ln_i_g,
           ln_i_b, W_halt, b_halt):
    ids = input_ids.reshape(-1)
    h0 = _sc_gather(embed, ids)
    logits, idx_pad = _fused(
        h0, W_e1, b_e1.reshape(1, -1), W_e2, b_e2.reshape(1, -1),
        ln_e_g.reshape(1, -1), ln_e_b.reshape(1, -1), W_idx,
        b_idx.reshape(1, -1), pool, W_i1, b_i1.reshape(1, -1), W_i2,
        b_i2.reshape(1, -1), ln_i_g.reshape(1, -1), ln_i_b.reshape(1, -1),
        W_halt, b_halt.reshape(1, -1), W_dec, b_dec.reshape(1, -1))
    logits = logits.reshape(_B, _T, _VOCAB)
    return (logits, (_NLOOP, idx_pad))


# speedup vs baseline: 1.0777x; 1.0777x over previous
import functools

import jax
import jax.numpy as jnp
from jax import lax
from jax.experimental import pallas as pl
from jax.experimental.pallas import tpu as pltpu
from jax.experimental.pallas import tpu_sc as plsc

_POOL_N = 500000
_MAX_K = 512
_NLOOP = 4
_HALT = 0.9
_D = 256
_VOCAB = 32000
_B = 4
_T = 512
_NTOK = _B * _T
_VB = 1280
_WIN = 640


def _layer_norm(x, g, b):
    m = jnp.mean(x, axis=-1, keepdims=True)
    v = jnp.mean((x - m) ** 2, axis=-1, keepdims=True)
    return (x - m) / jnp.sqrt(v + 1e-6) * g + b


def _softplus(x):
    return jnp.maximum(x, 0.0) + jnp.log(1.0 + jnp.exp(-jnp.abs(x)))


def _sc_gather(table, idx):
    info = plsc.get_sparse_core_info()
    nw = info.num_cores * info.num_subcores
    n = idx.shape[0]
    bpw = n // nw
    mesh = plsc.VectorSubcoreMesh(core_axis_name="c", subcore_axis_name="s")

    @functools.partial(
        pl.kernel,
        mesh=mesh,
        out_type=jax.ShapeDtypeStruct((n, _D), jnp.float32),
        scratch_types=[
            pltpu.VMEM((bpw,), jnp.int32),
            pltpu.VMEM((bpw, _D), jnp.float32),
            pltpu.SemaphoreType.DMA,
        ],
    )
    def k(table_hbm, idx_hbm, out_hbm, idx_v, rows_v, sem):
        wid = lax.axis_index("s") * info.num_cores + lax.axis_index("c")
        base = wid * bpw
        pltpu.sync_copy(idx_hbm.at[pl.ds(base, bpw)], idx_v)
        pltpu.async_copy(table_hbm.at[idx_v], rows_v, sem).wait()
        pltpu.sync_copy(rows_v, out_hbm.at[pl.ds(base, bpw)])

    return k(table, idx)


def _bdot(a, b):
    return jnp.dot(a.astype(jnp.bfloat16), b.astype(jnp.bfloat16),
                   preferred_element_type=jnp.float32)


def _fused_body(h0_ref, we1, be1, we2, be2, lneg, lneb, widx, bidx,
                pool_ref, wi1, bi1, wi2, bi2, lnig, lnib, whalt, bh,
                wdec_ref, bdec_ref, logits_ref, idx_out, state_ref,
                win_ref, idxs_ref, sem):
  j = pl.program_id(0)

  @pl.when(j == 0)
  def _controller_step():
    h0 = h0_ref[...]
    pre = _bdot(h0, we1[...]) + be1[...]
    h = h0 + _bdot(jax.nn.gelu(pre), we2[...]) + be2[...]
    h = _layer_norm(h, lneg[...], lneb[...])

    state = h
    halt_prob = jnp.zeros((_NTOK, 1), jnp.float32)
    halted = jnp.zeros((_NTOK, 1), jnp.float32)
    jvec = lax.broadcasted_iota(jnp.int32, (1, _WIN), 1)
    starts_list = []

    for _ in range(_NLOOP):
        pooled = jnp.concatenate(
            [jnp.mean(state[b * _T:(b + 1) * _T, :], axis=0, keepdims=True)
             for b in range(_B)],
            axis=0)
        raw = _bdot(pooled, widx[...]) + bidx[...]
        mu = jax.nn.sigmoid(raw[:, 0:1])
        sigma = _softplus(raw[:, 1:2]) + 1e-3
        start_i = jnp.floor(mu * float(_POOL_N - _MAX_K)).astype(jnp.int32)
        starts_list.append(start_i)
        astart = jnp.minimum((start_i // 8) * 8, _POOL_N - _WIN)
        off = start_i - astart
        row = jnp.concatenate([astart, jnp.zeros((_B, 127), jnp.int32)],
                              axis=1)
        idxs_ref[...] = jnp.concatenate(
            [row, jnp.zeros((8 - _B, 128), jnp.int32)], axis=0)
        copies = []
        for b in range(_B):
            a_b = pl.multiple_of(idxs_ref[b, 0], 8)
            c = pltpu.make_async_copy(
                pool_ref.at[pl.ds(a_b, _WIN), :], win_ref.at[b], sem)
            c.start()
            copies.append(c)
        retrieved = []
        for b in range(_B):
            copies[b].wait()
            sig = sigma[b:b + 1, 0:1]
            k = jvec - off[b:b + 1, 0:1]
            valid = (k >= 0) & (k < _MAX_K)
            pos = k.astype(jnp.float32) / float(_MAX_K) - 0.5
            wlog = -(pos * pos) / (2.0 * sig * sig)
            wmax = jnp.max(jnp.where(valid, wlog, -jnp.inf), axis=-1,
                           keepdims=True)
            e = jnp.where(valid, jnp.exp(wlog - wmax), 0.0)
            w = e / jnp.sum(e, axis=-1, keepdims=True)
            retrieved.append(_bdot(w, win_ref[b]))
        r_full = jnp.concatenate(
            [jnp.broadcast_to(retrieved[b], (_T, _D)) for b in range(_B)],
            axis=0)
        comb = jnp.concatenate([state, r_full], axis=1)
        integ = _bdot(jax.nn.gelu(_bdot(comb, wi1[...]) + bi1[...]),
                      wi2[...]) + bi2[...]
        integ = _layer_norm(integ, lnig[...], lnib[...])
        cand = state + integ
        p = jax.nn.sigmoid(_bdot(cand, whalt[...]) + bh[...])
        hp_new = halt_prob + p * (1.0 - halted)
        new_halted = (hp_new >= _HALT).astype(jnp.float32)
        state = (1.0 - halted) * cand + halted * state
        halt_prob = hp_new
        halted = new_halted

    state_ref[...] = state.astype(jnp.bfloat16)
    idx_out[...] = jnp.concatenate(starts_list, axis=1)

  logits_ref[...] = jnp.dot(state_ref[...], wdec_ref[...].astype(jnp.bfloat16),
                            preferred_element_type=jnp.float32) + bdec_ref[...]


def _fused(h0, W_e1, b_e1, W_e2, b_e2, ln_e_g, ln_e_b, W_idx, b_idx2,
           pool, W_i1, b_i1, W_i2, b_i2, ln_i_g, ln_i_b, W_halt, b_halt2,
           W_dec, b_dec2):
    hbm = pl.BlockSpec(memory_space=pltpu.MemorySpace.HBM)
    in_specs = [pl.BlockSpec(x.shape, lambda j: (0,) * x.ndim)
                for x in (h0, W_e1, b_e1, W_e2, b_e2, ln_e_g, ln_e_b,
                          W_idx, b_idx2)]
    in_specs.append(hbm)
    in_specs += [pl.BlockSpec(x.shape, lambda j: (0,) * x.ndim)
                 for x in (W_i1, b_i1, W_i2, b_i2, ln_i_g, ln_i_b,
                           W_halt, b_halt2)]
    in_specs += [pl.BlockSpec((_D, _VB), lambda j: (0, j)),
                 pl.BlockSpec((1, _VB), lambda j: (0, j))]
    return pl.pallas_call(
        _fused_body,
        grid=(_VOCAB // _VB,),
        in_specs=in_specs,
        out_specs=[pl.BlockSpec((_NTOK, _VB), lambda j: (0, j)),
                   pl.BlockSpec((_B, _NLOOP), lambda j: (0, 0))],
        out_shape=[jax.ShapeDtypeStruct((_NTOK, _VOCAB), jnp.float32),
                   jax.ShapeDtypeStruct((_B, _NLOOP), jnp.int32)],
        scratch_shapes=[pltpu.VMEM((_NTOK, _D), jnp.bfloat16),
                        pltpu.VMEM((_B, _WIN, _D), jnp.float32),
                        pltpu.VMEM((8, 128), jnp.int32),
                        pltpu.SemaphoreType.DMA],
    )(h0, W_e1, b_e1, W_e2, b_e2, ln_e_g, ln_e_b, W_idx, b_idx2, pool,
      W_i1, b_i1, W_i2, b_i2, ln_i_g, ln_i_b, W_halt, b_halt2, W_dec,
      b_dec2)


def kernel(input_ids, embed, W_e1, b_e1, W_e2, b_e2, ln_e_g, ln_e_b, W_dec,
           b_dec, W_idx, b_idx, pool, W_i1, b_i1, W_i2, b_i2, ln_i_g,
           ln_i_b, W_halt, b_halt):
    ids = input_ids.reshape(-1)
    h0 = _sc_gather(embed, ids)
    logits, idx_pad = _fused(
        h0, W_e1, b_e1.reshape(1, -1), W_e2, b_e2.reshape(1, -1),
        ln_e_g.reshape(1, -1), ln_e_b.reshape(1, -1), W_idx,
        b_idx.reshape(1, -1), pool, W_i1, b_i1.reshape(1, -1), W_i2,
        b_i2.reshape(1, -1), ln_i_g.reshape(1, -1), ln_i_b.reshape(1, -1),
        W_halt, b_halt.reshape(1, -1), W_dec, b_dec.reshape(1, -1))
    logits = logits.reshape(_B, _T, _VOCAB)
    return (logits, (_NLOOP, idx_pad))
